# baseline (speedup 1.0000x reference)
import functools

import jax
import jax.numpy as jnp
from jax import lax
from jax.experimental import pallas as pl
from jax.experimental.pallas import tpu as pltpu
from jax.experimental.pallas import tpu_sc as plsc

NEG_SLOPE = 0.2
K = 16
N = 8192
NUM_CLOUDS = 8
RB = 256
CB = 256
NB = N // RB
MB = 512
DP = 64
SPLIT = 1

TRUNC = -256
KBIAS = 0x00800000
KEY10 = 0x50150200 + KBIAS
EXCLKEY = 0x7F000000
MAXKEY = 0x7F700000
BIGI = 2**31 - 1


def _asf32(x):
    return lax.bitcast_convert_type(x, jnp.float32)


def _asi32(x):
    return lax.bitcast_convert_type(x, jnp.int32)


def _select_topk(topkey, topgi, cand, c0):
    maxf = _asf32(jnp.int32(MAXKEY))
    keys, gis = [], []
    for _ in range(K):
        mc = jnp.min(cand, axis=0, keepdims=True)
        mt = jnp.min(topkey, axis=0, keepdims=True)
        m = jnp.minimum(mc, mt)
        gi_c = c0 + (_asi32(m) & 0xFF)
        gi_t = jnp.min(jnp.where(topkey == m, topgi, BIGI), axis=0,
                       keepdims=True)
        gi = jnp.where(mc < mt, gi_c, gi_t)
        cand = jnp.where(cand == m, maxf, cand)
        topkey = jnp.where(topkey == m, maxf, topkey)
        keys.append(m)
        gis.append(gi)
    return jnp.concatenate(keys, axis=0), jnp.concatenate(gis, axis=0)


def _topk_kernel(blk_lo_ref, blk_hi_ref, rbat_ref, h_ref, hT_ref,
                 batc_ref, idx_ref, sqc_ref, sqr_ref, *, off):
    rb = pl.program_id(0) + off

    @pl.when(rb == off)
    def _():
        h2 = h_ref[...]
        h2 = h2 * h2
        ones_c = jnp.ones((1, h2.shape[1]), jnp.float32)
        sqc_ref[...] = lax.dot_general(
            h2, ones_c, (((1,), (1,)), ((), ())),
            preferred_element_type=jnp.float32,
            precision=lax.Precision.HIGHEST)
        hT2 = hT_ref[...]
        hT2 = hT2 * hT2
        sqr_ref[...] = lax.dot_general(
            ones_c, hT2, (((1,), (0,)), ((), ())),
            preferred_element_type=jnp.float32,
            precision=lax.Precision.HIGHEST)

    r0 = rb * RB
    rowsT = hT_ref[:, pl.ds(r0, RB)]
    rsq = sqr_ref[:, pl.ds(r0, RB)]
    rbat = rbat_ref[...]
    citer = lax.broadcasted_iota(jnp.int32, (CB, 1), 0)
    maxf = _asf32(jnp.int32(MAXKEY))
    key10 = _asf32(KEY10 + citer)
    exclf = _asf32(EXCLKEY + citer)

    topkey = jnp.full((K, RB), maxf, jnp.float32)
    topgi = jnp.full((K, RB), BIGI, jnp.int32)

    cbat0 = batc_ref[pl.ds(0, CB), :]
    cand0 = jnp.where(cbat0 == rbat, maxf, key10)
    topkey, topgi = _select_topk(topkey, topgi, cand0, 0)

    def cloud_body(cb, carry):
        topkey, topgi = carry
        c0 = cb * CB
        cols = h_ref[pl.ds(c0, CB), :]
        dots = lax.dot_general(cols, rowsT, (((1,), (0,)), ((), ())),
                               preferred_element_type=jnp.float32)
        csq = sqc_ref[pl.ds(c0, CB), :]
        dist = jnp.maximum(csq + rsq - 2.0 * dots, 0.0)
        key = _asf32((_asi32(dist) & TRUNC) + (KBIAS + citer))
        cbat = batc_ref[pl.ds(c0, CB), :]
        key = jnp.where(cbat == rbat, key, exclf)
        return _select_topk(topkey, topgi, key, c0)

    lo = blk_lo_ref[rb]
    hi = blk_hi_ref[rb]
    _, topgi = lax.fori_loop(lo, hi, cloud_body, (topkey, topgi))
    idx_ref[...] = topgi


def _topk(hp, hpT, rbatch2, cbatch2, blk_lo, blk_hi, part, nparts):
    d = hp.shape[1]
    nbp = NB // nparts
    off = part * nbp
    call = pl.pallas_call(
        functools.partial(_topk_kernel, off=off),
        grid_spec=pltpu.PrefetchScalarGridSpec(
            num_scalar_prefetch=2,
            grid=(nbp,),
            in_specs=[
                pl.BlockSpec((1, RB), lambda i, lo, hi: (0, i + off)),
                pl.BlockSpec((N, d), lambda i, lo, hi: (0, 0)),
                pl.BlockSpec((d, N), lambda i, lo, hi: (0, 0)),
                pl.BlockSpec((N, 1), lambda i, lo, hi: (0, 0)),
            ],
            out_specs=pl.BlockSpec((K, RB), lambda i, lo, hi: (0, i)),
            scratch_shapes=[pltpu.VMEM((N, 1), jnp.float32),
                            pltpu.VMEM((1, N), jnp.float32)],
        ),
        out_shape=jax.ShapeDtypeStruct((K, N // nparts), jnp.int32),
    )
    return call(blk_lo, blk_hi, cbatch2, hp, hpT, rbatch2)


_SC_NW = 32
_SC_CH = 128
_SC_NBUF = 6
_SC_LAG = 2


def _sc_gather(table, idx):
    B = idx.shape[0]
    D = table.shape[1]
    per_w = B // _SC_NW
    nch = per_w // _SC_CH
    mesh = plsc.VectorSubcoreMesh(core_axis_name="c", subcore_axis_name="s")

    @functools.partial(
        pl.kernel,
        out_type=jax.ShapeDtypeStruct((B, D), jnp.float32),
        mesh=mesh,
        compiler_params=pltpu.CompilerParams(use_tc_tiling_on_sc=False),
        scratch_types=[
            pltpu.VMEM((per_w,), jnp.int32),
            pltpu.VMEM((_SC_NBUF, _SC_CH, D), jnp.float32),
        ] + [pltpu.SemaphoreType.DMA] * (2 * _SC_NBUF),
    )
    def gather_k(table_hbm, idx_hbm, out_hbm, idx_v, rows, *sems):
        gsem = sems[:_SC_NBUF]
        ssem = sems[_SC_NBUF:]
        wid = lax.axis_index("s") * 2 + lax.axis_index("c")
        base = wid * per_w
        pltpu.sync_copy(idx_hbm.at[pl.ds(base, per_w)], idx_v)

        gh, sh = {}, {}

        def fire_store(g):
            b = g % _SC_NBUF
            sh[g] = pltpu.async_copy(
                rows.at[b], out_hbm.at[pl.ds(base + g * _SC_CH, _SC_CH)],
                ssem[b])

        for g in range(nch):
            b = g % _SC_NBUF
            if g >= _SC_NBUF:
                sh[g - _SC_NBUF].wait()
            gh[g] = pltpu.async_copy(
                table_hbm.at[idx_v.at[pl.ds(g * _SC_CH, _SC_CH)]],
                rows.at[b], gsem[b])
            if g >= _SC_LAG:
                gh[g - _SC_LAG].wait()
                fire_store(g - _SC_LAG)
        for g in range(nch - _SC_LAG, nch):
            gh[g].wait()
            fire_store(g)
        for g in range(nch - _SC_NBUF, nch):
            sh[g].wait()

    return gather_k(table, idx)


def _mlp_kernel(rows_ref, xj_ref, w1a_ref, w1b_ref, b1_ref, w2_ref, b2_ref,
                w3_ref, b3_ref, out_ref):
    rows = rows_ref[...]
    w1b = w1b_ref[...]
    w2 = w2_ref[...]
    w3 = w3_ref[...]
    b2 = b2_ref[...]
    b3 = b3_ref[...]
    base = jnp.dot(rows, w1a_ref[...],
                   preferred_element_type=jnp.float32) + b1_ref[...]
    acc = None
    for k in range(K):
        xjk = xj_ref[k]
        m = base + jnp.dot(xjk - rows, w1b,
                           preferred_element_type=jnp.float32)
        m = jnp.where(m >= 0, m, NEG_SLOPE * m)
        m = jnp.dot(m, w2, preferred_element_type=jnp.float32) + b2
        m = jnp.where(m >= 0, m, NEG_SLOPE * m)
        m = jnp.dot(m, w3, preferred_element_type=jnp.float32) + b3
        m = jnp.where(m >= 0, m, NEG_SLOPE * m)
        acc = m if acc is None else jnp.maximum(acc, m)
    out_ref[...] = acc


def _mlp(hp, xj3, w1a, w1b, b1, w2, b2, w3, b3, part, nparts):
    dout = w3.shape[1]
    np_rows = N // nparts
    moff = part * (np_rows // MB)
    call = pl.pallas_call(
        _mlp_kernel,
        grid=(np_rows // MB,),
        in_specs=[
            pl.BlockSpec((MB, DP), lambda i: (i + moff, 0)),
            pl.BlockSpec((K, MB, DP), lambda i: (0, i, 0)),
            pl.BlockSpec(w1a.shape, lambda i: (0, 0)),
            pl.BlockSpec(w1b.shape, lambda i: (0, 0)),
            pl.BlockSpec((1, w1a.shape[1]), lambda i: (0, 0)),
            pl.BlockSpec(w2.shape, lambda i: (0, 0)),
            pl.BlockSpec((1, w2.shape[1]), lambda i: (0, 0)),
            pl.BlockSpec(w3.shape, lambda i: (0, 0)),
            pl.BlockSpec((1, w3.shape[1]), lambda i: (0, 0)),
        ],
        out_specs=pl.BlockSpec((MB, dout), lambda i: (i, 0)),
        out_shape=jax.ShapeDtypeStruct((np_rows, dout), jnp.float32),
    )
    return call(hp, xj3, w1a, w1b, b1.reshape(1, -1), w2, b2.reshape(1, -1),
                w3, b3.reshape(1, -1))


def _block_bounds(batch):
    starts = jnp.searchsorted(
        batch, jnp.arange(NUM_CLOUDS + 1, dtype=batch.dtype), side="left"
    ).astype(jnp.int32)
    r = jnp.arange(NB, dtype=jnp.int32)
    lo_cloud = batch[r * RB]
    hi_cloud = batch[r * RB + (RB - 1)]
    col_lo = starts[lo_cloud]
    col_hi = starts[hi_cloud + 1]
    return col_lo // CB, (col_hi + CB - 1) // CB


def _edge_conv(h, rbatch2, cbatch2, blk_lo, blk_hi, Ws, bs):
    d = h.shape[1]
    w1 = Ws[0]
    if d < DP:
        hp = jnp.concatenate([h, jnp.zeros((N, DP - d), jnp.float32)], axis=1)
        z = jnp.zeros((DP - d, w1.shape[1]), jnp.float32)
        w1a = jnp.concatenate([w1[:d], z], axis=0)
        w1b = jnp.concatenate([w1[d:], z], axis=0)
    else:
        hp = h
        w1a = w1[:d]
        w1b = w1[d:]
    hpT = hp.T
    outs = []
    for p in range(SPLIT):
        idx = _topk(hp, hpT, rbatch2, cbatch2, blk_lo, blk_hi, p, SPLIT)
        xj = _sc_gather(hp, idx.reshape(-1))
        xj3 = xj.reshape(K, N // SPLIT, DP)
        outs.append(_mlp(hp, xj3, w1a, w1b, bs[0], Ws[1], bs[1],
                         Ws[2], bs[2], p, SPLIT))
    return jnp.concatenate(outs, axis=0)


def kernel(x, pos, batch, W11, b11, W12, b12, W13, b13, W21, b21, W22, b22,
           W23, b23, W31, b31, W32, b32, W33, b33):
    rbatch2 = batch[:, None]
    cbatch2 = batch[None, :]
    blk_lo, blk_hi = _block_bounds(batch)
    h = pos
    h = _edge_conv(h, rbatch2, cbatch2, blk_lo, blk_hi,
                   [W11, W12, W13], [b11, b12, b13])
    h = _edge_conv(h, rbatch2, cbatch2, blk_lo, blk_hi,
                   [W21, W22, W23], [b21, b22, b23])
    h = _edge_conv(h, rbatch2, cbatch2, blk_lo, blk_hi,
                   [W31, W32, W33], [b31, b32, b33])
    return h


# speedup vs baseline: 1.0692x; 1.0692x over previous
import functools

import jax
import jax.numpy as jnp
from jax import lax
from jax.experimental import pallas as pl
from jax.experimental.pallas import tpu as pltpu
from jax.experimental.pallas import tpu_sc as plsc

NEG_SLOPE = 0.2
K = 16
N = 8192
NUM_CLOUDS = 8
RB = 256
CB = 256
NB = N // RB
MB = 512
DP = 128
SPLIT = 1

TRUNC = -256
KBIAS = 0x00800000
KEY10 = 0x50150200 + KBIAS
EXCLKEY = 0x7F000000
MAXKEY = 0x7F700000
BIGI = 2**31 - 1


def _asf32(x):
    return lax.bitcast_convert_type(x, jnp.float32)


def _asi32(x):
    return lax.bitcast_convert_type(x, jnp.int32)


def _select_topk(topkey, topgi, cand, c0):
    maxf = _asf32(jnp.int32(MAXKEY))
    keys, gis = [], []
    for _ in range(K):
        mc = jnp.min(cand, axis=0, keepdims=True)
        mt = jnp.min(topkey, axis=0, keepdims=True)
        m = jnp.minimum(mc, mt)
        gi_c = c0 + (_asi32(m) & 0xFF)
        gi_t = jnp.min(jnp.where(topkey == m, topgi, BIGI), axis=0,
                       keepdims=True)
        gi = jnp.where(mc < mt, gi_c, gi_t)
        cand = jnp.where(cand == m, maxf, cand)
        topkey = jnp.where(topkey == m, maxf, topkey)
        keys.append(m)
        gis.append(gi)
    return jnp.concatenate(keys, axis=0), jnp.concatenate(gis, axis=0)


def _topk_kernel(blk_lo_ref, blk_hi_ref, rbat_ref, h_ref, hT_ref,
                 batc_ref, idx_ref, sqc_ref, sqr_ref, *, off):
    rb = pl.program_id(0) + off

    @pl.when(rb == off)
    def _():
        h2 = h_ref[...]
        h2 = h2 * h2
        ones_c = jnp.ones((1, h2.shape[1]), jnp.float32)
        sqc_ref[...] = lax.dot_general(
            h2, ones_c, (((1,), (1,)), ((), ())),
            preferred_element_type=jnp.float32,
            precision=lax.Precision.HIGHEST)
        hT2 = hT_ref[...]
        hT2 = hT2 * hT2
        sqr_ref[...] = lax.dot_general(
            ones_c, hT2, (((1,), (0,)), ((), ())),
            preferred_element_type=jnp.float32,
            precision=lax.Precision.HIGHEST)

    r0 = rb * RB
    rowsT = hT_ref[:, pl.ds(r0, RB)]
    rsq = sqr_ref[:, pl.ds(r0, RB)]
    rbat = rbat_ref[...]
    citer = lax.broadcasted_iota(jnp.int32, (CB, 1), 0)
    maxf = _asf32(jnp.int32(MAXKEY))
    key10 = _asf32(KEY10 + citer)
    exclf = _asf32(EXCLKEY + citer)

    topkey = jnp.full((K, RB), maxf, jnp.float32)
    topgi = jnp.full((K, RB), BIGI, jnp.int32)

    cbat0 = batc_ref[pl.ds(0, CB), :]
    cand0 = jnp.where(cbat0 == rbat, maxf, key10)
    topkey, topgi = _select_topk(topkey, topgi, cand0, 0)

    def cloud_body(cb, carry):
        topkey, topgi = carry
        c0 = cb * CB
        cols = h_ref[pl.ds(c0, CB), :]
        dots = lax.dot_general(cols, rowsT, (((1,), (0,)), ((), ())),
                               preferred_element_type=jnp.float32)
        csq = sqc_ref[pl.ds(c0, CB), :]
        dist = jnp.maximum(csq + rsq - 2.0 * dots, 0.0)
        key = _asf32((_asi32(dist) & TRUNC) + (KBIAS + citer))
        cbat = batc_ref[pl.ds(c0, CB), :]
        key = jnp.where(cbat == rbat, key, exclf)
        return _select_topk(topkey, topgi, key, c0)

    lo = blk_lo_ref[rb]
    hi = blk_hi_ref[rb]
    _, topgi = lax.fori_loop(lo, hi, cloud_body, (topkey, topgi))
    idx_ref[...] = topgi


def _topk(hp, hpT, rbatch2, cbatch2, blk_lo, blk_hi, part, nparts):
    d = hp.shape[1]
    nbp = NB // nparts
    off = part * nbp
    call = pl.pallas_call(
        functools.partial(_topk_kernel, off=off),
        grid_spec=pltpu.PrefetchScalarGridSpec(
            num_scalar_prefetch=2,
            grid=(nbp,),
            in_specs=[
                pl.BlockSpec((1, RB), lambda i, lo, hi: (0, i + off)),
                pl.BlockSpec((N, d), lambda i, lo, hi: (0, 0)),
                pl.BlockSpec((d, N), lambda i, lo, hi: (0, 0)),
                pl.BlockSpec((N, 1), lambda i, lo, hi: (0, 0)),
            ],
            out_specs=pl.BlockSpec((K, RB), lambda i, lo, hi: (0, i)),
            scratch_shapes=[pltpu.VMEM((N, 1), jnp.float32),
                            pltpu.VMEM((1, N), jnp.float32)],
        ),
        out_shape=jax.ShapeDtypeStruct((K, N // nparts), jnp.int32),
    )
    return call(blk_lo, blk_hi, cbatch2, hp, hpT, rbatch2)


_SC_NW = 32
_SC_CH = 128
_SC_NBUF = 6
_SC_LAG = 2


def _sc_gather(table, idx):
    B = idx.shape[0]
    D = table.shape[1]
    per_w = B // _SC_NW
    nch = per_w // _SC_CH
    mesh = plsc.VectorSubcoreMesh(core_axis_name="c", subcore_axis_name="s")

    @functools.partial(
        pl.kernel,
        out_type=jax.ShapeDtypeStruct((B, D), jnp.float32),
        mesh=mesh,
        scratch_types=[
            pltpu.VMEM((per_w,), jnp.int32),
            pltpu.VMEM((_SC_NBUF, _SC_CH, D), jnp.float32),
        ] + [pltpu.SemaphoreType.DMA] * (2 * _SC_NBUF),
    )
    def gather_k(table_hbm, idx_hbm, out_hbm, idx_v, rows, *sems):
        gsem = sems[:_SC_NBUF]
        ssem = sems[_SC_NBUF:]
        wid = lax.axis_index("s") * 2 + lax.axis_index("c")
        base = wid * per_w
        pltpu.sync_copy(idx_hbm.at[pl.ds(base, per_w)], idx_v)

        gh, sh = {}, {}

        def fire_store(g):
            b = g % _SC_NBUF
            sh[g] = pltpu.async_copy(
                rows.at[b], out_hbm.at[pl.ds(base + g * _SC_CH, _SC_CH)],
                ssem[b])

        for g in range(nch):
            b = g % _SC_NBUF
            if g >= _SC_NBUF:
                sh[g - _SC_NBUF].wait()
            gh[g] = pltpu.async_copy(
                table_hbm.at[idx_v.at[pl.ds(g * _SC_CH, _SC_CH)]],
                rows.at[b], gsem[b])
            if g >= _SC_LAG:
                gh[g - _SC_LAG].wait()
                fire_store(g - _SC_LAG)
        for g in range(nch - _SC_LAG, nch):
            gh[g].wait()
            fire_store(g)
        for g in range(nch - _SC_NBUF, nch):
            sh[g].wait()

    return gather_k(table, idx)


def _mlp_kernel(rows_ref, xj_ref, w1a_ref, w1b_ref, b1_ref, w2_ref, b2_ref,
                w3_ref, b3_ref, out_ref):
    rows = rows_ref[...]
    w1b = w1b_ref[...]
    w2 = w2_ref[...]
    w3 = w3_ref[...]
    b2 = b2_ref[...]
    b3 = b3_ref[...]
    base = jnp.dot(rows, w1a_ref[...],
                   preferred_element_type=jnp.float32) + b1_ref[...]
    acc = None
    for k in range(K):
        xjk = xj_ref[k]
        m = base + jnp.dot(xjk - rows, w1b,
                           preferred_element_type=jnp.float32)
        m = jnp.where(m >= 0, m, NEG_SLOPE * m)
        m = jnp.dot(m, w2, preferred_element_type=jnp.float32) + b2
        m = jnp.where(m >= 0, m, NEG_SLOPE * m)
        m = jnp.dot(m, w3, preferred_element_type=jnp.float32) + b3
        m = jnp.where(m >= 0, m, NEG_SLOPE * m)
        acc = m if acc is None else jnp.maximum(acc, m)
    out_ref[...] = acc


def _mlp(hp, xj3, w1a, w1b, b1, w2, b2, w3, b3, part, nparts):
    dout = w3.shape[1]
    np_rows = N // nparts
    moff = part * (np_rows // MB)
    call = pl.pallas_call(
        _mlp_kernel,
        grid=(np_rows // MB,),
        in_specs=[
            pl.BlockSpec((MB, DP), lambda i: (i + moff, 0)),
            pl.BlockSpec((K, MB, DP), lambda i: (0, i, 0)),
            pl.BlockSpec(w1a.shape, lambda i: (0, 0)),
            pl.BlockSpec(w1b.shape, lambda i: (0, 0)),
            pl.BlockSpec((1, w1a.shape[1]), lambda i: (0, 0)),
            pl.BlockSpec(w2.shape, lambda i: (0, 0)),
            pl.BlockSpec((1, w2.shape[1]), lambda i: (0, 0)),
            pl.BlockSpec(w3.shape, lambda i: (0, 0)),
            pl.BlockSpec((1, w3.shape[1]), lambda i: (0, 0)),
        ],
        out_specs=pl.BlockSpec((MB, dout), lambda i: (i, 0)),
        out_shape=jax.ShapeDtypeStruct((np_rows, dout), jnp.float32),
    )
    return call(hp, xj3, w1a, w1b, b1.reshape(1, -1), w2, b2.reshape(1, -1),
                w3, b3.reshape(1, -1))


def _block_bounds(batch):
    starts = jnp.searchsorted(
        batch, jnp.arange(NUM_CLOUDS + 1, dtype=batch.dtype), side="left"
    ).astype(jnp.int32)
    r = jnp.arange(NB, dtype=jnp.int32)
    lo_cloud = batch[r * RB]
    hi_cloud = batch[r * RB + (RB - 1)]
    col_lo = starts[lo_cloud]
    col_hi = starts[hi_cloud + 1]
    return col_lo // CB, (col_hi + CB - 1) // CB


def _edge_conv(h, rbatch2, cbatch2, blk_lo, blk_hi, Ws, bs):
    d = h.shape[1]
    w1 = Ws[0]
    if d < DP:
        hp = jnp.concatenate([h, jnp.zeros((N, DP - d), jnp.float32)], axis=1)
        z = jnp.zeros((DP - d, w1.shape[1]), jnp.float32)
        w1a = jnp.concatenate([w1[:d], z], axis=0)
        w1b = jnp.concatenate([w1[d:], z], axis=0)
    else:
        hp = h
        w1a = w1[:d]
        w1b = w1[d:]
    hpT = hp.T
    outs = []
    for p in range(SPLIT):
        idx = _topk(hp, hpT, rbatch2, cbatch2, blk_lo, blk_hi, p, SPLIT)
        xj = _sc_gather(hp, idx.reshape(-1))
        xj3 = xj.reshape(K, N // SPLIT, DP)
        outs.append(_mlp(hp, xj3, w1a, w1b, bs[0], Ws[1], bs[1],
                         Ws[2], bs[2], p, SPLIT))
    return jnp.concatenate(outs, axis=0)


def kernel(x, pos, batch, W11, b11, W12, b12, W13, b13, W21, b21, W22, b22,
           W23, b23, W31, b31, W32, b32, W33, b33):
    rbatch2 = batch[:, None]
    cbatch2 = batch[None, :]
    blk_lo, blk_hi = _block_bounds(batch)
    h = pos
    h = _edge_conv(h, rbatch2, cbatch2, blk_lo, blk_hi,
                   [W11, W12, W13], [b11, b12, b13])
    h = _edge_conv(h, rbatch2, cbatch2, blk_lo, blk_hi,
                   [W21, W22, W23], [b21, b22, b23])
    h = _edge_conv(h, rbatch2, cbatch2, blk_lo, blk_hi,
                   [W31, W32, W33], [b31, b32, b33])
    return h
